# Initial kernel scaffold; baseline (speedup 1.0000x reference)
import jax
import jax.numpy as jnp
from jax.experimental import pallas as pl


def kernel(xyz):
    raise NotImplementedError("write your pallas kernel here")


# speedup vs baseline: 9.0141x; 9.0141x over previous
import functools

import jax
import jax.numpy as jnp
from jax import lax
from jax.experimental import pallas as pl
from jax.experimental.pallas import tpu as pltpu
from jax.experimental.pallas import tpu_sc as plsc

_B, _N, _G, _K = 16, 16384, 1024, 32
_GW = _G // 2
_NV = _N // 16
_UNROLL = 8

_INF = float("inf")


def _fps_body(xt_ref, cent_ref, dist_ref):
    x = xt_ref[:, 0, :]
    y = xt_ref[:, 1, :]
    z = xt_ref[:, 2, :]
    lane = jax.lax.broadcasted_iota(jnp.int32, (_B, _N), 1)
    dist_ref[...] = jnp.full((_B, _N), 1e10, jnp.float32)

    def step(i, far):
        oh = (lane == far).astype(jnp.float32)
        cx = jnp.sum(x * oh, axis=1, keepdims=True)
        cy = jnp.sum(y * oh, axis=1, keepdims=True)
        cz = jnp.sum(z * oh, axis=1, keepdims=True)
        cent_ref[:, pl.ds(i, 1), :] = jnp.concatenate(
            [cx[:, :, None], cy[:, :, None], cz[:, :, None]], axis=2)
        dx = x - cx
        dy = y - cy
        dz = z - cz
        d = (dx * dx + dy * dy) + dz * dz
        dist = jnp.minimum(dist_ref[...], d)
        dist_ref[...] = dist
        m = jnp.max(dist, axis=1, keepdims=True)
        far_new = jnp.min(
            jnp.where(dist == m, lane, _N), axis=1, keepdims=True
        ).astype(jnp.int32)
        return far_new

    jax.lax.fori_loop(0, _G, step, jnp.zeros((_B, 1), jnp.int32))


def _fps(xt, interpret=False):
    return pl.pallas_call(
        _fps_body,
        out_shape=jax.ShapeDtypeStruct((_B, _G, 3), jnp.float32),
        scratch_shapes=[pltpu.VMEM((_B, _N), jnp.float32)],
        interpret=interpret,
    )(xt)


def _iota16():
    return lax.iota(jnp.int32, 16)


def _splat_f(ref, i):
    return plsc.load_gather(ref, [jnp.full((16,), i, jnp.int32)])


def _bfround(v):
    u = lax.bitcast_convert_type(v, jnp.uint32)
    one = jnp.uint32(1)
    r = (u + jnp.uint32(0x7FFF) + ((u >> jnp.uint32(16)) & one)) \
        & jnp.uint32(0xFFFF0000)
    return lax.bitcast_convert_type(r, jnp.float32)


def _cmp_sel(ak, av, bk, bv):
    sel = ak <= bk
    lo_k = jnp.where(sel, ak, bk)
    lo_v = jnp.where(sel, av, bv)
    hi_k = jnp.where(sel, bk, ak)
    hi_v = jnp.where(sel, bv, av)
    return lo_k, lo_v, hi_k, hi_v


def _rev(x):
    return lax.rev(x, dimensions=(0,))


def _sort32(k0, v0, k1, v1):
    k0, v0 = plsc.sort_key_val(k0, v0)
    k1, v1 = plsc.sort_key_val(k1, v1)
    lo_k, lo_v, hi_k, hi_v = _cmp_sel(k0, v0, _rev(k1), _rev(v1))
    lo_k, lo_v = plsc.sort_key_val(lo_k, lo_v)
    hi_k, hi_v = plsc.sort_key_val(hi_k, hi_v)
    return lo_k, lo_v, hi_k, hi_v


def _merge_low32(ak0, av0, ak1, av1, bk0, bv0, bk1, bv1):
    l0k, l0v, _, _ = _cmp_sel(ak0, av0, _rev(bk1), _rev(bv1))
    l1k, l1v, _, _ = _cmp_sel(ak1, av1, _rev(bk0), _rev(bv0))
    m0k, m0v, m1k, m1v = _cmp_sel(l0k, l0v, l1k, l1v)
    m0k, m0v = plsc.sort_key_val(m0k, m0v)
    m1k, m1v = plsc.sort_key_val(m1k, m1v)
    return m0k, m0v, m1k, m1v


def _knn_body(xt_hbm, cent_hbm, out_hbm,
              x_v, y_v, z_v, xn_v,
              xb_v, yb_v, zb_v,
              craw_v, cx_v, cy_v, cz_v, cn_v,
              cxb_v, cyb_v, czb_v, bm0_v, bm1_v,
              gid_v, pend_v, pendi_v, kept_v, kepti_v,
              stage_v, sem):
    del sem
    wid = lax.axis_index("s") * 2 + lax.axis_index("c")
    b = wid // 2
    h = wid % 2

    pltpu.sync_copy(xt_hbm.at[pl.ds(b * 3 * _N, _N)], x_v)
    pltpu.sync_copy(xt_hbm.at[pl.ds(b * 3 * _N + _N, _N)], y_v)
    pltpu.sync_copy(xt_hbm.at[pl.ds(b * 3 * _N + 2 * _N, _N)], z_v)
    pltpu.sync_copy(cent_hbm.at[pl.ds((b * _G + h * _GW) * 3, _GW * 3)], craw_v)

    it = _iota16()

    def xn_step(j, _):
        s = pl.ds(j * 16, 16)
        xv, yv, zv = x_v[s], y_v[s], z_v[s]
        xn_v[s] = (xv * xv + yv * yv) + zv * zv
        xb_v[s] = _bfround(xv)
        yb_v[s] = _bfround(yv)
        zb_v[s] = _bfround(zv)
        return 0

    lax.fori_loop(0, _NV, xn_step, 0)

    def csoa_step(j, _):
        rows = it + j * 16
        cx = plsc.load_gather(craw_v, [rows * 3])
        cy = plsc.load_gather(craw_v, [rows * 3 + 1])
        cz = plsc.load_gather(craw_v, [rows * 3 + 2])
        s = pl.ds(j * 16, 16)
        cx_v[s] = cx
        cy_v[s] = cy
        cz_v[s] = cz
        cn_v[s] = (cx * cx + cy * cy) + cz * cz
        cxb_v[s] = _bfround(cx)
        cyb_v[s] = _bfround(cy)
        czb_v[s] = _bfround(cz)
        return 0

    lax.fori_loop(0, _GW // 16, csoa_step, 0)

    bms = [bm0_v, bm1_v]

    def do_center_pair(p):
        g0 = p * 2
        cxs = [_splat_f(cx_v, g0), _splat_f(cx_v, g0 + 1)]
        cys = [_splat_f(cy_v, g0), _splat_f(cy_v, g0 + 1)]
        czs = [_splat_f(cz_v, g0), _splat_f(cz_v, g0 + 1)]
        cns = [_splat_f(cn_v, g0), _splat_f(cn_v, g0 + 1)]
        cxbs = [_splat_f(cxb_v, g0), _splat_f(cxb_v, g0 + 1)]
        cybs = [_splat_f(cyb_v, g0), _splat_f(cyb_v, g0 + 1)]
        czbs = [_splat_f(czb_v, g0), _splat_f(czb_v, g0 + 1)]

        def sq_at(i, s):
            xv, yv, zv, xnv = xb_v[s], yb_v[s], zb_v[s], xn_v[s]
            dot = (cxbs[i] * xv + cybs[i] * yv) + czbs[i] * zv
            return (cns[i] + xnv) - (jnp.float32(2.0) * dot)

        def dpass(jo, _):
            for u in range(_UNROLL):
                j = jo * _UNROLL + u
                s = pl.ds(j * 16, 16)
                for i in range(2):
                    sq = sq_at(i, s)
                    srt = lax.sort(sq)
                    plsc.store_compressed(bms[i].at[pl.ds(j, 16)], srt,
                                          mask=(it == 0))
            return 0

        lax.fori_loop(0, _NV // _UNROLL, dpass, 0)

        for i in range(2):
            def tau_step(j, carry):
                l1, l2 = carry
                bv = bms[i][pl.ds(j * 16, 16)]
                nl1 = jnp.minimum(l1, bv)
                nl2 = jnp.minimum(jnp.maximum(l1, bv), l2)
                return nl1, nl2

            _, l2 = lax.fori_loop(0, _NV // 16, tau_step,
                                  (jnp.full((16,), _INF),
                                   jnp.full((16,), _INF)))
            tau_v = jnp.full((16,), jnp.max(l2))

            def gid_step(j, goff):
                bv = bms[i][pl.ds(j * 16, 16)]
                m = bv <= tau_v
                cnt = jnp.sum(m.astype(jnp.int32))
                pos = goff + jnp.cumsum(m.astype(jnp.int32)) - 1
                plsc.store_scatter(gid_v, [pos], it + j * 16, mask=m)
                return goff + cnt

            ng = lax.fori_loop(0, _NV // 16, gid_step, jnp.int32(0))

            kept_v[pl.ds(0, 16)] = jnp.full((16,), _INF)
            kept_v[pl.ds(16, 16)] = jnp.full((16,), _INF)
            kepti_v[pl.ds(0, 16)] = jnp.zeros((16,), jnp.int32)
            kepti_v[pl.ds(16, 16)] = jnp.zeros((16,), jnp.int32)

            def drain_chunk():
                c0k, c0i = pend_v[pl.ds(0, 16)], pendi_v[pl.ds(0, 16)]
                c1k, c1i = pend_v[pl.ds(16, 16)], pendi_v[pl.ds(16, 16)]
                c0k, c0i, c1k, c1i = _sort32(c0k, c0i, c1k, c1i)
                a0k, a0i = kept_v[pl.ds(0, 16)], kepti_v[pl.ds(0, 16)]
                a1k, a1i = kept_v[pl.ds(16, 16)], kepti_v[pl.ds(16, 16)]
                m0k, m0i, m1k, m1i = _merge_low32(a0k, a0i, a1k, a1i,
                                                  c0k, c0i, c1k, c1i)
                kept_v[pl.ds(0, 16)] = m0k
                kept_v[pl.ds(16, 16)] = m1k
                kepti_v[pl.ds(0, 16)] = m0i
                kepti_v[pl.ds(16, 16)] = m1i
                pend_v[pl.ds(0, 16)] = pend_v[pl.ds(32, 16)]
                pendi_v[pl.ds(0, 16)] = pendi_v[pl.ds(32, 16)]

            def cand_step(g, coff):
                gid = gid_v[pl.ds(g, 16)][0]
                dv = sq_at(i, pl.ds(gid * 16, 16))
                m = dv <= tau_v
                cnt = jnp.sum(m.astype(jnp.int32))
                pos = coff + jnp.cumsum(m.astype(jnp.int32)) - 1
                plsc.store_scatter(pend_v, [pos], dv, mask=m)
                plsc.store_scatter(pendi_v, [pos], it + gid * 16, mask=m)
                coff = coff + cnt

                def drain_and_adjust():
                    drain_chunk()
                    return coff - 32

                return lax.cond(coff >= 32, drain_and_adjust, lambda: coff)

            coff = lax.fori_loop(0, ng, cand_step, jnp.int32(0))

            p0 = pend_v[pl.ds(0, 16)]
            p1 = pend_v[pl.ds(16, 16)]
            pend_v[pl.ds(0, 16)] = jnp.where(it < coff, p0, _INF)
            pend_v[pl.ds(16, 16)] = jnp.where(it + 16 < coff, p1, _INF)
            drain_chunk()

            slot = (p % 4) * 2 + i
            for half in range(2):
                idxv = kepti_v[pl.ds(half * 16, 16)]
                gx = plsc.load_gather(x_v, [idxv]) - cxs[i]
                gy = plsc.load_gather(y_v, [idxv]) - cys[i]
                gz = plsc.load_gather(z_v, [idxv]) - czs[i]
                pos = jnp.full((16,), slot * 96 + half * 48, jnp.int32) + it * 3
                plsc.store_scatter(stage_v, [pos], gx)
                plsc.store_scatter(stage_v, [pos + 1], gy)
                plsc.store_scatter(stage_v, [pos + 2], gz)

    def block_step(blk, _):
        for q in range(4):
            do_center_pair(blk * 4 + q)
        g_base = (b * _G + h * _GW + blk * 8) * _K * 3
        pltpu.sync_copy(stage_v, out_hbm.at[pl.ds(g_base, 8 * _K * 3)])
        return 0

    lax.fori_loop(0, _GW // 8, block_step, 0)


def _knn_sc(xt1, cent1):
    mesh = plsc.VectorSubcoreMesh(core_axis_name="c", subcore_axis_name="s")
    f = functools.partial(
        pl.kernel,
        out_type=jax.ShapeDtypeStruct((_B * _G * _K * 3,), jnp.float32),
        mesh=mesh,
        compiler_params=pltpu.CompilerParams(needs_layout_passes=False),
        scratch_types=[
            pltpu.VMEM((_N,), jnp.float32),
            pltpu.VMEM((_N,), jnp.float32),
            pltpu.VMEM((_N,), jnp.float32),
            pltpu.VMEM((_N,), jnp.float32),
            pltpu.VMEM((_N,), jnp.float32),
            pltpu.VMEM((_N,), jnp.float32),
            pltpu.VMEM((_N,), jnp.float32),
            pltpu.VMEM((_GW * 3,), jnp.float32),
            pltpu.VMEM((_GW,), jnp.float32),
            pltpu.VMEM((_GW,), jnp.float32),
            pltpu.VMEM((_GW,), jnp.float32),
            pltpu.VMEM((_GW,), jnp.float32),
            pltpu.VMEM((_GW,), jnp.float32),
            pltpu.VMEM((_GW,), jnp.float32),
            pltpu.VMEM((_GW,), jnp.float32),
            pltpu.VMEM((_NV + 16,), jnp.float32),
            pltpu.VMEM((_NV + 16,), jnp.float32),
            pltpu.VMEM((_NV + 16,), jnp.int32),
            pltpu.VMEM((64,), jnp.float32),
            pltpu.VMEM((64,), jnp.int32),
            pltpu.VMEM((32,), jnp.float32),
            pltpu.VMEM((32,), jnp.int32),
            pltpu.VMEM((8 * _K * 3,), jnp.float32),
            pltpu.SemaphoreType.DMA,
        ],
    )(_knn_body)
    return f(xt1, cent1)


def kernel(xyz):
    xt = xyz.transpose(0, 2, 1)
    center = _fps(xt)
    nb = _knn_sc(xt.reshape(-1), center.reshape(-1))
    return (nb.reshape(_B, _G, _K, 3), center)


# speedup vs baseline: 17.7846x; 1.9730x over previous
import functools

import jax
import jax.numpy as jnp
from jax import lax
from jax.experimental import pallas as pl
from jax.experimental.pallas import tpu as pltpu
from jax.experimental.pallas import tpu_sc as plsc

_B, _N, _G, _K = 16, 16384, 1024, 32
_GW = _G // 2
_NV = _N // 16
_UNROLL = 8

_INF = float("inf")


def _fps_body(xt_ref, cent_ref, dist_ref):
    x = xt_ref[:, 0, :]
    y = xt_ref[:, 1, :]
    z = xt_ref[:, 2, :]
    lane = jax.lax.broadcasted_iota(jnp.int32, (_B, _N), 1)
    dist_ref[...] = jnp.full((_B, _N), 1e10, jnp.float32)

    def step(i, far):
        oh = (lane == far).astype(jnp.float32)
        cx = jnp.sum(x * oh, axis=1, keepdims=True)
        cy = jnp.sum(y * oh, axis=1, keepdims=True)
        cz = jnp.sum(z * oh, axis=1, keepdims=True)
        cent_ref[:, pl.ds(i, 1), :] = jnp.concatenate(
            [cx[:, :, None], cy[:, :, None], cz[:, :, None]], axis=2)
        dx = x - cx
        dy = y - cy
        dz = z - cz
        d = (dx * dx + dy * dy) + dz * dz
        dist = jnp.minimum(dist_ref[...], d)
        dist_ref[...] = dist
        m = jnp.max(dist, axis=1, keepdims=True)
        far_new = jnp.min(
            jnp.where(dist == m, lane, _N), axis=1, keepdims=True
        ).astype(jnp.int32)
        return far_new

    jax.lax.fori_loop(0, _G, step, jnp.zeros((_B, 1), jnp.int32))


def _fps(xt, interpret=False):
    return pl.pallas_call(
        _fps_body,
        out_shape=jax.ShapeDtypeStruct((_B, _G, 3), jnp.float32),
        scratch_shapes=[pltpu.VMEM((_B, _N), jnp.float32)],
        interpret=interpret,
    )(xt)


def _iota16():
    return lax.iota(jnp.int32, 16)


def _splat_f(ref, i):
    return plsc.load_gather(ref, [jnp.full((16,), i, jnp.int32)])


def _bfround(v):
    u = lax.bitcast_convert_type(v, jnp.uint32)
    one = jnp.uint32(1)
    r = (u + jnp.uint32(0x7FFF) + ((u >> jnp.uint32(16)) & one)) \
        & jnp.uint32(0xFFFF0000)
    return lax.bitcast_convert_type(r, jnp.float32)


def _cmp_sel(ak, av, bk, bv):
    sel = ak <= bk
    lo_k = jnp.where(sel, ak, bk)
    lo_v = jnp.where(sel, av, bv)
    hi_k = jnp.where(sel, bk, ak)
    hi_v = jnp.where(sel, bv, av)
    return lo_k, lo_v, hi_k, hi_v


def _rev(x):
    return lax.rev(x, dimensions=(0,))


def _sort32(k0, v0, k1, v1):
    k0, v0 = plsc.sort_key_val(k0, v0)
    k1, v1 = plsc.sort_key_val(k1, v1)
    lo_k, lo_v, hi_k, hi_v = _cmp_sel(k0, v0, _rev(k1), _rev(v1))
    lo_k, lo_v = plsc.sort_key_val(lo_k, lo_v)
    hi_k, hi_v = plsc.sort_key_val(hi_k, hi_v)
    return lo_k, lo_v, hi_k, hi_v


def _merge_low32(ak0, av0, ak1, av1, bk0, bv0, bk1, bv1):
    l0k, l0v, _, _ = _cmp_sel(ak0, av0, _rev(bk1), _rev(bv1))
    l1k, l1v, _, _ = _cmp_sel(ak1, av1, _rev(bk0), _rev(bv0))
    m0k, m0v, m1k, m1v = _cmp_sel(l0k, l0v, l1k, l1v)
    m0k, m0v = plsc.sort_key_val(m0k, m0v)
    m1k, m1v = plsc.sort_key_val(m1k, m1v)
    return m0k, m0v, m1k, m1v


def _knn_body(xt_hbm, cent_hbm, out_hbm,
              x_v, y_v, z_v, xn_v,
              xb_v, yb_v, zb_v,
              craw_v, cx_v, cy_v, cz_v, cn_v,
              cxb_v, cyb_v, czb_v, bm0_v, bm1_v,
              gid_v, pend_v, pendi_v, kept_v, kepti_v,
              stage_v, sem):
    del sem
    wid = lax.axis_index("s") * 2 + lax.axis_index("c")
    b = wid // 2
    h = wid % 2

    pltpu.sync_copy(xt_hbm.at[pl.ds(b * 3 * _N, _N)], x_v)
    pltpu.sync_copy(xt_hbm.at[pl.ds(b * 3 * _N + _N, _N)], y_v)
    pltpu.sync_copy(xt_hbm.at[pl.ds(b * 3 * _N + 2 * _N, _N)], z_v)
    pltpu.sync_copy(cent_hbm.at[pl.ds((b * _G + h * _GW) * 3, _GW * 3)], craw_v)

    it = _iota16()

    def xn_step(j, _):
        s = pl.ds(j * 16, 16)
        xv, yv, zv = x_v[s], y_v[s], z_v[s]
        xn_v[s] = (xv * xv + yv * yv) + zv * zv
        xb_v[s] = _bfround(xv)
        yb_v[s] = _bfround(yv)
        zb_v[s] = _bfround(zv)
        return 0

    lax.fori_loop(0, _NV, xn_step, 0)

    def csoa_step(j, _):
        rows = it + j * 16
        cx = plsc.load_gather(craw_v, [rows * 3])
        cy = plsc.load_gather(craw_v, [rows * 3 + 1])
        cz = plsc.load_gather(craw_v, [rows * 3 + 2])
        s = pl.ds(j * 16, 16)
        cx_v[s] = cx
        cy_v[s] = cy
        cz_v[s] = cz
        cn_v[s] = (cx * cx + cy * cy) + cz * cz
        cxb_v[s] = _bfround(cx)
        cyb_v[s] = _bfround(cy)
        czb_v[s] = _bfround(cz)
        return 0

    lax.fori_loop(0, _GW // 16, csoa_step, 0)

    bms = [bm0_v, bm1_v]

    def do_center_pair(p):
        g0 = p * 2
        cxs = [_splat_f(cx_v, g0), _splat_f(cx_v, g0 + 1)]
        cys = [_splat_f(cy_v, g0), _splat_f(cy_v, g0 + 1)]
        czs = [_splat_f(cz_v, g0), _splat_f(cz_v, g0 + 1)]
        cns = [_splat_f(cn_v, g0), _splat_f(cn_v, g0 + 1)]
        cxbs = [_splat_f(cxb_v, g0), _splat_f(cxb_v, g0 + 1)]
        cybs = [_splat_f(cyb_v, g0), _splat_f(cyb_v, g0 + 1)]
        czbs = [_splat_f(czb_v, g0), _splat_f(czb_v, g0 + 1)]

        def sq_at(i, s):
            xv, yv, zv, xnv = xb_v[s], yb_v[s], zb_v[s], xn_v[s]
            dot = (cxbs[i] * xv + cybs[i] * yv) + czbs[i] * zv
            return (cns[i] + xnv) - (jnp.float32(2.0) * dot)

        @plsc.parallel_loop(0, _NV, unroll=_UNROLL)
        def dpass(j):
            s = pl.ds(j * 16, 16)
            for i in range(2):
                sq = sq_at(i, s)
                srt = lax.sort(sq)
                plsc.store_compressed(bms[i].at[pl.ds(j, 16)], srt,
                                      mask=(it == 0))

        for i in range(2):
            def tau_step(j, carry):
                l1, l2 = carry
                bv = bms[i][pl.ds(j * 16, 16)]
                nl1 = jnp.minimum(l1, bv)
                nl2 = jnp.minimum(jnp.maximum(l1, bv), l2)
                return nl1, nl2

            _, l2 = lax.fori_loop(0, _NV // 16, tau_step,
                                  (jnp.full((16,), _INF),
                                   jnp.full((16,), _INF)))
            tau_v = jnp.full((16,), jnp.max(l2))

            def gid_step(j, goff):
                bv = bms[i][pl.ds(j * 16, 16)]
                m = bv <= tau_v
                cnt = jnp.sum(m.astype(jnp.int32))
                pos = goff + jnp.cumsum(m.astype(jnp.int32)) - 1
                plsc.store_scatter(gid_v, [pos], it + j * 16, mask=m)
                return goff + cnt

            ng = lax.fori_loop(0, _NV // 16, gid_step, jnp.int32(0))

            kept_v[pl.ds(0, 16)] = jnp.full((16,), _INF)
            kept_v[pl.ds(16, 16)] = jnp.full((16,), _INF)
            kepti_v[pl.ds(0, 16)] = jnp.zeros((16,), jnp.int32)
            kepti_v[pl.ds(16, 16)] = jnp.zeros((16,), jnp.int32)

            def drain_chunk():
                c0k, c0i = pend_v[pl.ds(0, 16)], pendi_v[pl.ds(0, 16)]
                c1k, c1i = pend_v[pl.ds(16, 16)], pendi_v[pl.ds(16, 16)]
                c0k, c0i, c1k, c1i = _sort32(c0k, c0i, c1k, c1i)
                a0k, a0i = kept_v[pl.ds(0, 16)], kepti_v[pl.ds(0, 16)]
                a1k, a1i = kept_v[pl.ds(16, 16)], kepti_v[pl.ds(16, 16)]
                m0k, m0i, m1k, m1i = _merge_low32(a0k, a0i, a1k, a1i,
                                                  c0k, c0i, c1k, c1i)
                kept_v[pl.ds(0, 16)] = m0k
                kept_v[pl.ds(16, 16)] = m1k
                kepti_v[pl.ds(0, 16)] = m0i
                kepti_v[pl.ds(16, 16)] = m1i
                pend_v[pl.ds(0, 16)] = pend_v[pl.ds(32, 16)]
                pendi_v[pl.ds(0, 16)] = pendi_v[pl.ds(32, 16)]

            def cand_step(g, coff):
                gid = gid_v[pl.ds(g, 16)][0]
                dv = sq_at(i, pl.ds(gid * 16, 16))
                m = dv <= tau_v
                cnt = jnp.sum(m.astype(jnp.int32))
                pos = coff + jnp.cumsum(m.astype(jnp.int32)) - 1
                plsc.store_scatter(pend_v, [pos], dv, mask=m)
                plsc.store_scatter(pendi_v, [pos], it + gid * 16, mask=m)
                coff = coff + cnt

                def drain_and_adjust():
                    drain_chunk()
                    return coff - 32

                return lax.cond(coff >= 32, drain_and_adjust, lambda: coff)

            coff = lax.fori_loop(0, ng, cand_step, jnp.int32(0))

            p0 = pend_v[pl.ds(0, 16)]
            p1 = pend_v[pl.ds(16, 16)]
            pend_v[pl.ds(0, 16)] = jnp.where(it < coff, p0, _INF)
            pend_v[pl.ds(16, 16)] = jnp.where(it + 16 < coff, p1, _INF)
            drain_chunk()

            slot = (p % 4) * 2 + i
            for half in range(2):
                idxv = kepti_v[pl.ds(half * 16, 16)]
                gx = plsc.load_gather(x_v, [idxv]) - cxs[i]
                gy = plsc.load_gather(y_v, [idxv]) - cys[i]
                gz = plsc.load_gather(z_v, [idxv]) - czs[i]
                pos = jnp.full((16,), slot * 96 + half * 48, jnp.int32) + it * 3
                plsc.store_scatter(stage_v, [pos], gx)
                plsc.store_scatter(stage_v, [pos + 1], gy)
                plsc.store_scatter(stage_v, [pos + 2], gz)

    def block_step(blk, _):
        for q in range(4):
            do_center_pair(blk * 4 + q)
        g_base = (b * _G + h * _GW + blk * 8) * _K * 3
        pltpu.sync_copy(stage_v, out_hbm.at[pl.ds(g_base, 8 * _K * 3)])
        return 0

    lax.fori_loop(0, _GW // 8, block_step, 0)


def _knn_sc(xt1, cent1):
    mesh = plsc.VectorSubcoreMesh(core_axis_name="c", subcore_axis_name="s")
    f = functools.partial(
        pl.kernel,
        out_type=jax.ShapeDtypeStruct((_B * _G * _K * 3,), jnp.float32),
        mesh=mesh,
        compiler_params=pltpu.CompilerParams(needs_layout_passes=False),
        scratch_types=[
            pltpu.VMEM((_N,), jnp.float32),
            pltpu.VMEM((_N,), jnp.float32),
            pltpu.VMEM((_N,), jnp.float32),
            pltpu.VMEM((_N,), jnp.float32),
            pltpu.VMEM((_N,), jnp.float32),
            pltpu.VMEM((_N,), jnp.float32),
            pltpu.VMEM((_N,), jnp.float32),
            pltpu.VMEM((_GW * 3,), jnp.float32),
            pltpu.VMEM((_GW,), jnp.float32),
            pltpu.VMEM((_GW,), jnp.float32),
            pltpu.VMEM((_GW,), jnp.float32),
            pltpu.VMEM((_GW,), jnp.float32),
            pltpu.VMEM((_GW,), jnp.float32),
            pltpu.VMEM((_GW,), jnp.float32),
            pltpu.VMEM((_GW,), jnp.float32),
            pltpu.VMEM((_NV + 16,), jnp.float32),
            pltpu.VMEM((_NV + 16,), jnp.float32),
            pltpu.VMEM((_NV + 16,), jnp.int32),
            pltpu.VMEM((64,), jnp.float32),
            pltpu.VMEM((64,), jnp.int32),
            pltpu.VMEM((32,), jnp.float32),
            pltpu.VMEM((32,), jnp.int32),
            pltpu.VMEM((8 * _K * 3,), jnp.float32),
            pltpu.SemaphoreType.DMA,
        ],
    )(_knn_body)
    return f(xt1, cent1)


def kernel(xyz):
    xt = xyz.transpose(0, 2, 1)
    center = _fps(xt)
    nb = _knn_sc(xt.reshape(-1), center.reshape(-1))
    return (nb.reshape(_B, _G, _K, 3), center)
